# baseline (speedup 1.0000x reference)
import jax
import jax.numpy as jnp
from jax import lax
from jax.experimental import pallas as pl
from jax.experimental.pallas import tpu as pltpu
from jax.experimental.pallas import tpu_sc as plsc

_BATCH = 1024
_ND = 100
_NS = 20
_NCH = 1000
_IDX_PAD = 112
_DATA = _ND * _NS
_DATA_PAD = 2016
_OUT_W = _NCH * _NS
_DENSE_PAD = 20032
_NCORES = 2
_NSUB = 16
_NW = _NCORES * _NSUB
_RPW = _BATCH // _NW
_L = 16


def _body(idx_hbm, data_hbm, out_hbm, idx_v, data_v, dense_v, sem):
    cid = lax.axis_index("c")
    sid = lax.axis_index("s")
    wid = sid * _NCORES + cid

    lane = lax.iota(jnp.int32, _L)
    mask_tail = lane < (_NS - _L)
    zeros = jnp.zeros((_L,), jnp.float32)

    def _z(i, carry):
        dense_v[pl.ds(i * _L, _L)] = zeros
        return carry
    lax.fori_loop(0, _DENSE_PAD // _L, _z, 0)

    def _row(i, carry):
        r = wid * _RPW + i
        pltpu.sync_copy(idx_hbm.at[r], idx_v)
        pltpu.sync_copy(data_hbm.at[r], data_v)
        for c in range(7):
            chunk = idx_v[pl.ds(c * _L, _L)]
            for j in range(_L if c < 6 else _ND - 6 * _L):
                d = c * _L + j
                base = chunk[j]
                v1 = data_v[pl.ds(d * _NS, _L)]
                v2 = jnp.where(mask_tail,
                               data_v[pl.ds(d * _NS + _L, _L)], 0.0)
                plsc.addupdate(dense_v.at[pl.ds(base, _L)], v1)
                plsc.addupdate(dense_v.at[pl.ds(base + _L, _L)], v2)
        pltpu.sync_copy(dense_v.at[pl.ds(0, _OUT_W)], out_hbm.at[r])
        for c in range(7):
            chunk = idx_v[pl.ds(c * _L, _L)]
            for j in range(_L if c < 6 else _ND - 6 * _L):
                base = chunk[j]
                dense_v[pl.ds(base, _L)] = zeros
                dense_v[pl.ds(base + (_NS - _L), _L)] = zeros
        return carry

    lax.fori_loop(0, _RPW, _row, 0)


def kernel(inputs):
    idx = (inputs[:, :_ND].astype(jnp.int32) * _NS)
    idx = jnp.pad(idx, ((0, 0), (0, _IDX_PAD - _ND)))
    data = jnp.pad(inputs[:, _ND:], ((0, 0), (0, _DATA_PAD - _DATA)))
    mesh = plsc.VectorSubcoreMesh(
        core_axis_name="c", subcore_axis_name="s",
        num_cores=_NCORES, num_subcores=_NSUB)
    run = pl.kernel(
        _body,
        out_type=jax.ShapeDtypeStruct((_BATCH, _OUT_W), jnp.float32),
        mesh=mesh,
        compiler_params=pltpu.CompilerParams(
            use_tc_tiling_on_sc=False, needs_layout_passes=False),
        scratch_types=[
            pltpu.VMEM((_IDX_PAD,), jnp.int32),
            pltpu.VMEM((_DATA_PAD,), jnp.float32),
            pltpu.VMEM((_DENSE_PAD,), jnp.float32),
            pltpu.SemaphoreType.DMA,
        ],
    )
    out = run(idx, data)
    return out.reshape(_BATCH, _NCH, _NS)[..., None]


# speedup vs baseline: 1.0632x; 1.0632x over previous
import jax
import jax.numpy as jnp
from jax import lax
from jax.experimental import pallas as pl
from jax.experimental.pallas import tpu as pltpu
from jax.experimental.pallas import tpu_sc as plsc

_BATCH = 1024
_ND = 100
_NS = 20
_NCH = 1000
_ROW = _ND + _ND * _NS
_OUT_W = _NCH * _NS
_NCORES = 2
_NSUB = 16
_NW = _NCORES * _NSUB
_RPW = _BATCH // _NW
_L = 16


def _body(in_hbm, out_hbm, rows_v, dense_v, sem):
    cid = lax.axis_index("c")
    sid = lax.axis_index("s")
    wid = sid * _NCORES + cid

    lane = lax.iota(jnp.int32, _L)
    mask_head = lane < (_L - (_NS - _L))
    zeros = jnp.zeros((_L,), jnp.float32)

    def _z(i, carry):
        dense_v[pl.ds(i * _L, _L)] = zeros
        return carry
    lax.fori_loop(0, _OUT_W // _L, _z, 0)

    pltpu.sync_copy(in_hbm.at[pl.ds(wid * _RPW, _RPW)], rows_v)

    def _row(i, carry):
        for c in range(7):
            chunk = (rows_v[i, pl.ds(c * _L, _L)]).astype(jnp.int32) * _NS
            for j in range(_L if c < 6 else _ND - 6 * _L):
                d = c * _L + j
                base = chunk[j]
                v1 = rows_v[i, pl.ds(_ND + d * _NS, _L)]
                w = rows_v[i, pl.ds(_ND + d * _NS + (_NS - _L), _L)]
                v2 = jnp.where(mask_head, 0.0, w)
                plsc.addupdate(dense_v.at[pl.ds(base, _L)], v1)
                plsc.addupdate(dense_v.at[pl.ds(base + (_NS - _L), _L)], v2)
        pltpu.sync_copy(dense_v, out_hbm.at[wid * _RPW + i])
        for c in range(7):
            chunk = (rows_v[i, pl.ds(c * _L, _L)]).astype(jnp.int32) * _NS
            for j in range(_L if c < 6 else _ND - 6 * _L):
                base = chunk[j]
                dense_v[pl.ds(base, _L)] = zeros
                dense_v[pl.ds(base + (_NS - _L), _L)] = zeros
        return carry

    lax.fori_loop(0, _RPW, _row, 0)


def kernel(inputs):
    mesh = plsc.VectorSubcoreMesh(
        core_axis_name="c", subcore_axis_name="s",
        num_cores=_NCORES, num_subcores=_NSUB)
    run = pl.kernel(
        _body,
        out_type=jax.ShapeDtypeStruct((_BATCH, _OUT_W), jnp.float32),
        mesh=mesh,
        compiler_params=pltpu.CompilerParams(
            use_tc_tiling_on_sc=False, needs_layout_passes=False),
        scratch_types=[
            pltpu.VMEM((_RPW, _ROW), jnp.float32),
            pltpu.VMEM((_OUT_W,), jnp.float32),
            pltpu.SemaphoreType.DMA,
        ],
    )
    out = run(inputs)
    return out.reshape(_BATCH, _NCH, _NS)[..., None]


# speedup vs baseline: 3.3514x; 3.1520x over previous
import jax
import jax.numpy as jnp
from jax import lax
from jax.experimental import pallas as pl
from jax.experimental.pallas import tpu as pltpu
from jax.experimental.pallas import tpu_sc as plsc

_BATCH = 1024
_ND = 100
_NS = 20
_NCH = 1000
_ROW = _ND + _ND * _NS
_NCORES = 2
_NSUB = 16
_NW = _NCORES * _NSUB
_L = 16
_BPW = _BATCH // _NW
_NPASS = 4
_CPP = _NCH // _NPASS
_PROWS = _CPP * _NS


def _body(xt_hbm, out_hbm, xblk_v, dense_v, sem):
    cid = lax.axis_index("c")
    sid = lax.axis_index("s")
    wid = sid * _NCORES + cid

    lane = lax.iota(jnp.int32, _L)
    zeros = jnp.zeros((_L,), jnp.float32)

    for g in range(2):
        b0 = (wid * _BPW + g * _L).astype(jnp.int32)
        pltpu.sync_copy(xt_hbm.at[:, pl.ds(b0, _L)], xblk_v)
        for p in range(_NPASS):
            def _z(i, carry):
                for k in range(_NS):
                    dense_v[i, k, 0, :] = zeros
                return carry
            lax.fori_loop(0, _CPP, _z, 0)

            zero_i = jnp.zeros((_L,), jnp.int32)

            def _scan(d, carry):
                ch = xblk_v[d, :].astype(jnp.int32) - (p * _CPP)
                m = (ch >= 0) & (ch < _CPP)
                for s in range(_NS):
                    v = xblk_v[_ND + d * _NS + s, :]
                    plsc.addupdate_scatter(
                        dense_v, [ch, zero_i + s, zero_i, lane], v, mask=m)
                return carry
            lax.fori_loop(0, _ND, _scan, 0)

            pltpu.sync_copy(
                dense_v,
                out_hbm.at[pl.ds(p * _CPP, _CPP), :, :, pl.ds(b0, _L)])


def kernel(inputs):
    xt = inputs.T
    mesh = plsc.VectorSubcoreMesh(
        core_axis_name="c", subcore_axis_name="s",
        num_cores=_NCORES, num_subcores=_NSUB)
    run = pl.kernel(
        _body,
        out_type=jax.ShapeDtypeStruct((_NCH, _NS, 1, _BATCH), jnp.float32),
        mesh=mesh,
        compiler_params=pltpu.CompilerParams(
            use_tc_tiling_on_sc=False, needs_layout_passes=False),
        scratch_types=[
            pltpu.VMEM((_ROW, _L), jnp.float32),
            pltpu.VMEM((_CPP, _NS, 1, _L), jnp.float32),
            pltpu.SemaphoreType.DMA,
        ],
    )
    out = run(xt)
    return out.transpose(3, 0, 1, 2)
